# baseline (speedup 1.0000x reference)
import functools

import jax
import jax.numpy as jnp
from jax.experimental import pallas as pl
from jax.experimental.pallas import tpu as pltpu


def _fused_kernel(src_ref, dst_ref, nb_ref, a_ref, a2_ref, x_ref, w_ref,
                  b_ref, out_ref, ta_s, tb_s, p_tile, *, n_tiles):
    i = pl.program_id(0)
    tm = a_ref.shape[0]
    te = out_ref.shape[0]

    @pl.when(i < n_tiles)
    def _sage_step():
        nh = a_ref.shape[1]
        neigh = (jnp.dot(a_ref[...], x_ref[pl.ds(0, nh), :],
                         preferred_element_type=jnp.float32) +
                 jnp.dot(a2_ref[...], x_ref[pl.ds(nh, nh), :],
                         preferred_element_type=jnp.float32))
        xs = x_ref[pl.ds(i * tm, tm), :]
        xz = jnp.concatenate([xs, neigh], axis=1)
        h = jnp.dot(xz, w_ref[...], preferred_element_type=jnp.float32)
        h = jnp.maximum(h + b_ref[...], 0.0)

        nb = nb_ref[pl.ds(i * tm, tm), :]
        lane = jax.lax.broadcasted_iota(jnp.int32, (tm, 128), 1)
        zeros = jnp.zeros((tm, 128), jnp.float32)
        ones = jnp.ones((tm, 128), jnp.float32)
        ea = jnp.where(lane == 0, nb, jnp.where(lane == 1, ones, zeros))
        eb = jnp.where(lane == 0, ones, jnp.where(lane == 1, nb, zeros))
        rows = pl.ds(i * tm, tm)
        ta_s[rows] = jnp.concatenate([h[:, None, :], ea[:, None, :]], axis=1)
        tb_s[rows] = jnp.concatenate([h[:, None, :], eb[:, None, :]], axis=1)

    @pl.when(i >= n_tiles)
    def _edge_step():
        base = (i - n_tiles) * te
        for mi in range(te):
            s = src_ref[base + mi]
            d = dst_ref[base + mi]
            p_tile[mi] = ta_s[s] * tb_s[d]
        prod = p_tile[...]
        half = prod[:, 0, :] + prod[:, 1, :]
        out_ref[...] = jnp.sum(half, axis=1, keepdims=True)


def _fused(x, a_norm, w_stacked, b, nb_col, src, dst, *, tm, te):
    n, din = x.shape
    d = w_stacked.shape[1]
    e = src.shape[0]
    n_tiles = n // tm
    e_tiles = e // te

    flops = 2 * n * n * din + 2 * n * (2 * din) * d + 6 * e * 128
    bytes_accessed = 4 * (n * n + n * din + 2 * din * d + d + n + e * 3)

    out = pl.pallas_call(
        functools.partial(_fused_kernel, n_tiles=n_tiles),
        out_shape=jax.ShapeDtypeStruct((e, 1), jnp.float32),
        grid_spec=pltpu.PrefetchScalarGridSpec(
            num_scalar_prefetch=2,
            grid=(n_tiles + e_tiles,),
            in_specs=[
                pl.BlockSpec((n, 1), lambda i, s, dd: (0, 0)),
                pl.BlockSpec((tm, n // 2),
                             lambda i, s, dd, t=n_tiles: (jnp.minimum(i, t - 1), 0)),
                pl.BlockSpec((tm, n // 2),
                             lambda i, s, dd, t=n_tiles: (jnp.minimum(i, t - 1), 1)),
                pl.BlockSpec((n, din), lambda i, s, dd: (0, 0)),
                pl.BlockSpec((2 * din, d), lambda i, s, dd: (0, 0)),
                pl.BlockSpec((1, d), lambda i, s, dd: (0, 0)),
            ],
            out_specs=pl.BlockSpec(
                (te, 1), lambda i, s, dd, t=n_tiles: (jnp.maximum(i - t, 0), 0)),
            scratch_shapes=[
                pltpu.VMEM((n, 2, 128), jnp.float32),
                pltpu.VMEM((n, 2, 128), jnp.float32),
                pltpu.VMEM((te, 2, 128), jnp.float32),
            ],
        ),
        compiler_params=pltpu.CompilerParams(
            dimension_semantics=("arbitrary",)),
        cost_estimate=pl.CostEstimate(flops=flops, transcendentals=0,
                                      bytes_accessed=bytes_accessed),
    )(src, dst, nb_col, a_norm, a_norm, x, w_stacked, b)
    return out.reshape(e)


def kernel(x, a_norm, w_self, w_neigh, sage_bias, node_biases, src, dst):
    n, din = x.shape
    w_stacked = jnp.concatenate([w_self, w_neigh], axis=0)
    nb_col = node_biases[1:].reshape(n, 1).astype(jnp.float32)
    e = src.shape[0]

    tm = 512 if n % 4096 == 0 else n // 2
    te = 1024 if e % 4096 == 0 else e // 2
    return _fused(x, a_norm, w_stacked, sage_bias, nb_col,
                  src.astype(jnp.int32), dst.astype(jnp.int32), tm=tm, te=te)


# speedup vs baseline: 1.0592x; 1.0592x over previous
import functools

import jax
import jax.numpy as jnp
from jax.experimental import pallas as pl
from jax.experimental.pallas import tpu as pltpu


def _fused_kernel(src_ref, dst_ref, nb_ref, a_ref, x_ref, ws_ref, wn_ref,
                  b_ref, out_ref, ta_s, tb_s, p_tile, *, n_tiles):
    i = pl.program_id(0)
    tm = a_ref.shape[0]
    te = out_ref.shape[0]

    @pl.when(i < n_tiles)
    def _sage_step():
        neigh = jnp.dot(a_ref[...], x_ref[...],
                        preferred_element_type=jnp.float32)
        xs = x_ref[pl.ds(i * tm, tm), :]
        h = (jnp.dot(xs, ws_ref[...], preferred_element_type=jnp.float32) +
             jnp.dot(neigh, wn_ref[...], preferred_element_type=jnp.float32))
        h = jnp.maximum(h + b_ref[...], 0.0)

        nb = nb_ref[pl.ds(i * tm, tm), :]
        lane = jax.lax.broadcasted_iota(jnp.int32, (tm, 128), 1)
        zeros = jnp.zeros((tm, 128), jnp.float32)
        ones = jnp.ones((tm, 128), jnp.float32)
        ea = jnp.where(lane == 0, nb, jnp.where(lane == 1, ones, zeros))
        eb = jnp.where(lane == 0, ones, jnp.where(lane == 1, nb, zeros))
        rows = pl.ds(i * tm, tm)
        ta_s[rows] = jnp.concatenate([h[:, None, :], ea[:, None, :]], axis=1)
        tb_s[rows] = jnp.concatenate([h[:, None, :], eb[:, None, :]], axis=1)

    @pl.when(i >= n_tiles)
    def _edge_step():
        base = (i - n_tiles) * te
        for mi in range(te):
            s = src_ref[base + mi]
            d = dst_ref[base + mi]
            p_tile[mi] = ta_s[s] * tb_s[d]
        prod = p_tile[...]
        half = prod[:, 0, :] + prod[:, 1, :]
        out_ref[...] = jnp.sum(half, axis=1, keepdims=True)


def _fused(x, a_norm, w_self, w_neigh, b, nb_col, src, dst, *, tm, te):
    n, din = x.shape
    d = w_self.shape[1]
    e = src.shape[0]
    n_tiles = n // tm
    e_tiles = e // te

    flops = 2 * n * n * din + 2 * n * (2 * din) * d + 6 * e * 128
    bytes_accessed = 4 * (n * n + n * din + 2 * din * d + d + n + e * 3)

    out = pl.pallas_call(
        functools.partial(_fused_kernel, n_tiles=n_tiles),
        out_shape=jax.ShapeDtypeStruct((e, 1), jnp.float32),
        grid_spec=pltpu.PrefetchScalarGridSpec(
            num_scalar_prefetch=2,
            grid=(n_tiles + e_tiles,),
            in_specs=[
                pl.BlockSpec((n, 1), lambda i, s, dd: (0, 0)),
                pl.BlockSpec((tm, n),
                             lambda i, s, dd, t=n_tiles: (jnp.minimum(i, t - 1), 0)),
                pl.BlockSpec((n, din), lambda i, s, dd: (0, 0)),
                pl.BlockSpec((din, d), lambda i, s, dd: (0, 0)),
                pl.BlockSpec((din, d), lambda i, s, dd: (0, 0)),
                pl.BlockSpec((1, d), lambda i, s, dd: (0, 0)),
            ],
            out_specs=pl.BlockSpec(
                (te, 1), lambda i, s, dd, t=n_tiles: (jnp.maximum(i - t, 0), 0)),
            scratch_shapes=[
                pltpu.VMEM((n, 2, 128), jnp.float32),
                pltpu.VMEM((n, 2, 128), jnp.float32),
                pltpu.VMEM((te, 2, 128), jnp.float32),
            ],
        ),
        compiler_params=pltpu.CompilerParams(
            dimension_semantics=("arbitrary",)),
        cost_estimate=pl.CostEstimate(flops=flops, transcendentals=0,
                                      bytes_accessed=bytes_accessed),
    )(src, dst, nb_col, a_norm, x, w_self, w_neigh, b)
    return out.reshape(e)


def kernel(x, a_norm, w_self, w_neigh, sage_bias, node_biases, src, dst):
    n, din = x.shape
    nb_col = node_biases[1:].reshape(n, 1).astype(jnp.float32)
    e = src.shape[0]

    tm = 512 if n % 4096 == 0 else n // 2
    te = 1024 if e % 4096 == 0 else e // 2
    return _fused(x, a_norm, w_self, w_neigh, sage_bias, nb_col,
                  src.astype(jnp.int32), dst.astype(jnp.int32), tm=tm, te=te)
